# baseline (speedup 1.0000x reference)
import functools

import jax
import jax.numpy as jnp
from jax import lax
from jax.experimental import pallas as pl
from jax.experimental.pallas import tpu as pltpu
from jax.experimental.pallas import tpu_sc as plsc

HIDDEN = 128
EPS = 1e-12

NC, NS = 2, 16
NW = NC * NS
B_TOTAL = 4096 * 200
B_PER_W = B_TOTAL // NW
G = 128
NG = B_PER_W // G
REP = 128
NB = 4
NR = NG // NB


def _rsqrt16(v):
    bits = lax.bitcast_convert_type(v, jnp.int32)
    y = lax.bitcast_convert_type(
        jnp.int32(0x5F3759DF) - lax.shift_right_logical(bits, 1), jnp.float32)
    for _ in range(3):
        y = y * (1.5 - 0.5 * v * y * y)
    return y


def _gather_body(idx_hbm, table_hbm, gamma_hbm, beta_hbm, out_hbm,
                 idx_v, rows_v, tab_v, gb_v, nt_v, spm_table,
                 gsem, ssem, isem):
    cid = lax.axis_index("c")
    sid = lax.axis_index("s")
    wid = sid * NC + cid
    base = wid * B_PER_W

    pltpu.async_copy(idx_hbm.at[wid], idx_v, isem)

    pltpu.sync_copy(table_hbm, tab_v)
    pltpu.sync_copy(gamma_hbm, gb_v.at[0])
    pltpu.sync_copy(beta_hbm, gb_v.at[1])
    lanes16 = lax.iota(jnp.int32, 16)

    dnums = lax.GatherDimensionNumbers(
        offset_dims=(), collapsed_slice_dims=(0,), start_index_map=(0,))

    def shuffle(v, idx):
        return lax.gather(v, idx[:, None], dnums, slice_sizes=(1,),
                          mode=lax.GatherScatterMode.PROMISE_IN_BOUNDS)

    def hsum(v):
        for sh in (8, 4, 2, 1):
            v = v + shuffle(v, lax.bitwise_xor(lanes16, sh))
        return v

    for s in range(4):
        x = [tab_v[s, pl.ds(c * 16, 16)] for c in range(8)]
        tot = x[0]
        for c in range(1, 8):
            tot = tot + x[c]
        mean = hsum(tot) * (1.0 / HIDDEN)
        cx = [xc - mean for xc in x]
        sq = cx[0] * cx[0]
        for c in range(1, 8):
            sq = sq + cx[c] * cx[c]
        var = hsum(sq) * (1.0 / HIDDEN)
        y = _rsqrt16(var + EPS)
        for c in range(8):
            sl = pl.ds(c * 16, 16)
            n = cx[c] * y * gb_v[0, sl] + gb_v[1, sl]
            nt_v[s, sl] = n
            nt_v[s + 4, sl] = n

    for k in range(4):
        pltpu.sync_copy(nt_v, spm_table.at[pl.ds(8 * (sid * 4 + k), 8)])
    plsc.subcore_barrier()

    lanes = lax.iota(jnp.int32, 16)

    def fix_row(t):
        for c in range(8):
            sl = pl.ds(c * 16, 16)
            idx_v[t, sl] = idx_v[t, sl] + (lanes * 4 + c * 64)

    def gstart(b, j):
        pltpu.async_copy(spm_table.at[idx_v.at[j]], rows_v.at[b], gsem.at[b])

    def gwait(b, j):
        pltpu.make_async_copy(
            spm_table.at[idx_v.at[j]], rows_v.at[b], gsem.at[b]).wait()

    def sstart(b, j):
        pltpu.async_copy(
            rows_v.at[b], out_hbm.at[pl.ds(base + j * G, G)], ssem.at[b])

    def swait(b, j):
        pltpu.make_async_copy(
            rows_v.at[b], out_hbm.at[pl.ds(base + j * G, G)], ssem.at[b]).wait()

    pltpu.make_async_copy(idx_hbm.at[wid], idx_v, isem).wait()

    for b in range(NB):
        fix_row(b)
        gstart(b, b)

    def round_(r, _):
        j0 = r * NB
        for b in range(NB):
            j = j0 + b
            fix_row(j + NB)
            gwait(b, j)
            sstart(b, j)
            swait(b, j)
            gstart(b, j + NB)
        return 0

    lax.fori_loop(0, NR - 1, round_, 0)

    j0 = (NR - 1) * NB
    for b in range(NB):
        gwait(b, j0 + b)
        sstart(b, j0 + b)
        swait(b, j0 + b)


_gather = functools.partial(
    pl.kernel,
    out_type=jax.ShapeDtypeStruct((B_TOTAL, HIDDEN), jnp.float32),
    mesh=plsc.VectorSubcoreMesh(
        core_axis_name="c", subcore_axis_name="s", num_cores=NC, num_subcores=NS
    ),
    scratch_types=[
        pltpu.VMEM((NG, G), jnp.int32),
        pltpu.VMEM((NB, G, HIDDEN), jnp.float32),
        pltpu.VMEM((4, HIDDEN), jnp.float32),
        pltpu.VMEM((2, HIDDEN), jnp.float32),
        pltpu.VMEM((8, HIDDEN), jnp.float32),
        pltpu.VMEM_SHARED((4 * REP, HIDDEN), jnp.float32),
        pltpu.SemaphoreType.DMA((NB,)),
        pltpu.SemaphoreType.DMA((NB,)),
        pltpu.SemaphoreType.DMA,
    ],
)(_gather_body)


def kernel(input_rotation, table, gamma, beta):
    idx = input_rotation.reshape(NW, NG, G).astype(jnp.int32)
    out = _gather(idx, table, gamma, beta)
    return out.reshape(4096, 200, HIDDEN)


# speedup vs baseline: 1.0114x; 1.0114x over previous
import functools

import jax
import jax.numpy as jnp
from jax import lax
from jax.experimental import pallas as pl
from jax.experimental.pallas import tpu as pltpu
from jax.experimental.pallas import tpu_sc as plsc

HIDDEN = 128
EPS = 1e-12

NC, NS = 2, 16
NW = NC * NS
B_TOTAL = 4096 * 200
B_PER_W = B_TOTAL // NW
G = 128
NG = B_PER_W // G
REP = 32
NB = 4
NR = NG // NB


def _ln_body(t_ref, g_ref, b_ref, o_ref):
    t = t_ref[...]
    mean = jnp.mean(t, axis=-1, keepdims=True)
    c = t - mean
    var = jnp.mean(c * c, axis=-1, keepdims=True)
    n = c * lax.rsqrt(var + EPS) * g_ref[...] + b_ref[...]
    o_ref[...] = jnp.broadcast_to(n[None], (REP, 4, HIDDEN))


def _normed_table_rep(table, gamma, beta):
    out = pl.pallas_call(
        _ln_body,
        out_shape=jax.ShapeDtypeStruct((REP, 4, HIDDEN), jnp.float32),
    )(table, gamma.reshape(1, HIDDEN), beta.reshape(1, HIDDEN))
    return out.reshape(4 * REP, HIDDEN)


def _gather_body(idx_hbm, table_hbm, out_hbm, idx_v, rows_v, spm_table,
                 gsem, ssem, isem):
    sid = lax.axis_index("s")
    wid = sid * NC + lax.axis_index("c")
    base = wid * B_PER_W
    pltpu.sync_copy(
        table_hbm.at[pl.ds(sid * (4 * REP // NS), 4 * REP // NS)],
        spm_table.at[pl.ds(sid * (4 * REP // NS), 4 * REP // NS)])

    pltpu.sync_copy(idx_hbm.at[wid, pl.ds(0, 8)], idx_v.at[pl.ds(0, 8)])
    pltpu.async_copy(
        idx_hbm.at[wid, pl.ds(8, NG - 8)], idx_v.at[pl.ds(8, NG - 8)], isem)
    plsc.subcore_barrier()

    lanes = lax.iota(jnp.int32, 16)

    def fix_row(t):
        for c in range(8):
            sl = pl.ds(c * 16, 16)
            idx_v[t, sl] = idx_v[t, sl] + (lanes * 4 + (c % 2) * 64)

    def gstart(b, j):
        pltpu.async_copy(spm_table.at[idx_v.at[j]], rows_v.at[b], gsem.at[b])

    def gwait(b, j):
        pltpu.make_async_copy(
            spm_table.at[idx_v.at[j]], rows_v.at[b], gsem.at[b]).wait()

    def sstart(b, j):
        pltpu.async_copy(
            rows_v.at[b], out_hbm.at[pl.ds(base + j * G, G)], ssem.at[b])

    def swait(b, j):
        pltpu.make_async_copy(
            rows_v.at[b], out_hbm.at[pl.ds(base + j * G, G)], ssem.at[b]).wait()

    for b in range(NB):
        fix_row(b)
        gstart(b, b)

    pltpu.make_async_copy(
        idx_hbm.at[wid, pl.ds(8, NG - 8)], idx_v.at[pl.ds(8, NG - 8)],
        isem).wait()

    def round_(r, _):
        j0 = r * NB
        for b in range(NB):
            j = j0 + b
            fix_row(j + NB)
            gwait(b, j)
            sstart(b, j)
            swait(b, j)
            gstart(b, j + NB)
        return 0

    lax.fori_loop(0, NR - 1, round_, 0)

    j0 = (NR - 1) * NB
    for b in range(NB):
        gwait(b, j0 + b)
        sstart(b, j0 + b)
        swait(b, j0 + b)


_gather = functools.partial(
    pl.kernel,
    out_type=jax.ShapeDtypeStruct((B_TOTAL, HIDDEN), jnp.float32),
    mesh=plsc.VectorSubcoreMesh(
        core_axis_name="c", subcore_axis_name="s", num_cores=NC, num_subcores=NS
    ),
    scratch_types=[
        pltpu.VMEM((NG, G), jnp.int32),
        pltpu.VMEM((NB, G, HIDDEN), jnp.float32),
        pltpu.VMEM_SHARED((4 * REP, HIDDEN), jnp.float32),
        pltpu.SemaphoreType.DMA((NB,)),
        pltpu.SemaphoreType.DMA((NB,)),
        pltpu.SemaphoreType.DMA,
    ],
)(_gather_body)


def kernel(input_rotation, table, gamma, beta):
    table_rep = _normed_table_rep(table, gamma, beta)
    idx = input_rotation.reshape(NW, NG, G).astype(jnp.int32)
    out = _gather(idx, table_rep)
    return out.reshape(4096, 200, HIDDEN)
